# baseline (speedup 1.0000x reference)
import functools

import numpy as np

import jax
import jax.numpy as jnp
from jax import lax
from jax.experimental import pallas as pl
from jax.experimental.pallas import tpu as pltpu
from jax.experimental.pallas import tpu_sc as plsc

N = 10000
E = 320000
H = 128
OUT_DIM = 2

NC = 2
NS = 16
NW = NC * NS
CH = 80
ROWS_PER_SUB = 632
NPAD = NS * ROWS_PER_SUB

K = 5
ECH = E // K
EPW = ECH // NW
NCHUNK = EPW // CH

NB = 1000
EB = 4000

_f32 = jnp.float32
_bf16 = jnp.bfloat16

_PERM = np.empty((H,), np.int32)
for _g in range(H // 32):
    for _i in range(16):
        _PERM[32 * _g + 2 * _i] = 32 * _g + _i
        _PERM[32 * _g + 2 * _i + 1] = 32 * _g + 16 + _i


def _ln(x, g, b):
    m = jnp.mean(x, axis=-1, keepdims=True)
    v = jnp.mean((x - m) ** 2, axis=-1, keepdims=True)
    return (x - m) * lax.rsqrt(v + 1e-5) * g + b


def _dot(a, b):
    return jnp.dot(a, b, preferred_element_type=_f32)


def _bdot(a, b):
    return jnp.dot(a.astype(_bf16), b, preferred_element_type=_f32)


def _node_enc_body(x, w0, b0, w1, b1, h_o):
    h_o[...] = _dot(jax.nn.relu(_dot(x[...], w0[...]) + b0[...]), w1[...]) + b1[...]


def _node_enc(x, w0, b0, w1, b1):
    grid = N // NB
    blk = pl.BlockSpec((NB, H), lambda i: (i, 0))
    wspec = pl.BlockSpec((H, H), lambda i: (0, 0))
    bspec = pl.BlockSpec((1, H), lambda i: (0, 0))
    return pl.pallas_call(
        _node_enc_body,
        grid=(grid,),
        in_specs=[blk, wspec, bspec, wspec, bspec],
        out_specs=blk,
        out_shape=jax.ShapeDtypeStruct((N, H), _f32),
    )(x, w0, b0, w1, b1)


def _edge_core(e0, hdb, hsb, wa, wb, wd, wc, be0, we1, be1, wg, bn0, wn1,
               bn1):
    u = (_dot(hdb, wa[...]) + _dot(hsb, wb[...]) + _bdot(e0, wc[...])
         + be0[...])
    e_new = _bdot(jax.nn.relu(u), we1[...]) + be1[...]
    v = _dot(hdb, wd[...]) + _bdot(e_new, wg[...]) + bn0[...]
    msg = _bdot(jax.nn.relu(v), wn1[...]) + bn1[...]
    return e_new, msg


def _edge_body_l0(ea, hd, hs, ew0, eb0, ew1, eb1, wa, wb, wd, wc, be0, we1,
                  be1, wg, bn0, wn1, bn1, eng, enb, e_o, msg_o):
    e0 = _bdot(jax.nn.relu(_bdot(ea[...], ew0[...]) + eb0[...]), ew1[...]) \
        + eb1[...]
    e_new, msg = _edge_core(e0, hd[...], hs[...], wa, wb, wd, wc, be0, we1,
                            be1, wg, bn0, wn1, bn1)
    e_o[...] = _ln(e0 + e_new, eng[...], enb[...]).astype(_bf16)
    msg_o[...] = msg.astype(_bf16)


def _edge_body_l1(e, hd, hs, wa, wb, wd, wc, be0, we1, be1, wg, bn0, wn1,
                  bn1, msg_o):
    _, msg = _edge_core(e[...].astype(_f32), hd[...], hs[...], wa, wb, wd,
                        wc, be0, we1, be1, wg, bn0, wn1, bn1)
    msg_o[...] = msg.astype(_bf16)


def _edge_l0(c, ea_full, hd, hs, ew0, eb0, ew1, eb1, wa, wb, wd, wc, be0,
             we1, be1, wg, bn0, wn1, bn1, eng, enb):
    grid = ECH // EB
    d_edge = ea_full.shape[1]
    eablk = pl.BlockSpec((EB, d_edge), lambda i: (c * (ECH // EB) + i, 0))
    eblk = pl.BlockSpec((EB, H), lambda i: (i, 0))
    wspec = pl.BlockSpec((H, H), lambda i: (0, 0))
    bspec = pl.BlockSpec((1, H), lambda i: (0, 0))
    return pl.pallas_call(
        _edge_body_l0,
        grid=(grid,),
        in_specs=[eablk, eblk, eblk,
                  pl.BlockSpec((d_edge, H), lambda i: (0, 0)), bspec,
                  wspec, bspec,
                  wspec, wspec, wspec, wspec, bspec, wspec, bspec, wspec,
                  bspec, wspec, bspec, bspec, bspec],
        out_specs=[eblk, eblk],
        out_shape=[jax.ShapeDtypeStruct((ECH, H), _bf16),
                   jax.ShapeDtypeStruct((ECH, H), _bf16)],
    )(ea_full, hd, hs, ew0, eb0, ew1, eb1, wa, wb, wd, wc, be0, we1, be1,
      wg, bn0, wn1, bn1, eng, enb)


def _edge_l1(e, hd, hs, wa, wb, wd, wc, be0, we1, be1, wg, bn0, wn1, bn1):
    grid = ECH // EB
    eblk = pl.BlockSpec((EB, H), lambda i: (i, 0))
    wspec = pl.BlockSpec((H, H), lambda i: (0, 0))
    bspec = pl.BlockSpec((1, H), lambda i: (0, 0))
    return pl.pallas_call(
        _edge_body_l1,
        grid=(grid,),
        in_specs=[eblk, eblk, eblk, wspec, wspec, wspec, wspec, bspec,
                  wspec, bspec, wspec, bspec, wspec, bspec],
        out_specs=eblk,
        out_shape=jax.ShapeDtypeStruct((ECH, H), _bf16),
    )(e, hd, hs, wa, wb, wd, wc, be0, we1, be1, wg, bn0, wn1, bn1)


def _node_upd_body(h, s0, s1, g, b, h_o):
    h_o[...] = _ln(h[...] + s0[0] + s1[0], g[...], b[...])


def _node_update(h, s_parts, g, b):
    grid = N // NB
    blk = pl.BlockSpec((NB, H), lambda i: (i, 0))
    sblk = pl.BlockSpec((1, NB, H), lambda i: (0, i, 0))
    bspec = pl.BlockSpec((1, H), lambda i: (0, 0))
    return pl.pallas_call(
        _node_upd_body,
        grid=(grid,),
        in_specs=[blk, sblk, sblk, bspec, bspec],
        out_specs=blk,
        out_shape=jax.ShapeDtypeStruct((N, H), _f32),
    )(h, s_parts[0:1], s_parts[1:2], g, b)


def _node_dec_body(h, s0, s1, g, b, w0, b0, w1, b1, out_o):
    hn = _ln(h[...] + s0[0] + s1[0], g[...], b[...])
    out_o[...] = _dot(jax.nn.relu(_dot(hn, w0[...]) + b0[...]), w1[...]) + b1[...]


def _node_decode(h, s_parts, g, b, w0, b0, w1, b1):
    grid = N // NB
    blk = pl.BlockSpec((NB, H), lambda i: (i, 0))
    sblk = pl.BlockSpec((1, NB, H), lambda i: (0, i, 0))
    wspec = pl.BlockSpec((H, H), lambda i: (0, 0))
    bspec = pl.BlockSpec((1, H), lambda i: (0, 0))
    return pl.pallas_call(
        _node_dec_body,
        grid=(grid,),
        in_specs=[blk, sblk, sblk, bspec, bspec, wspec, bspec,
                  pl.BlockSpec((H, OUT_DIM), lambda i: (0, 0)),
                  pl.BlockSpec((1, OUT_DIM), lambda i: (0, 0))],
        out_specs=pl.BlockSpec((NB, OUT_DIM), lambda i: (i, 0)),
        out_shape=jax.ShapeDtypeStruct((N, OUT_DIM), _f32),
    )(h, s_parts[0:1], s_parts[1:2], g, b, w0, b0, w1, b1)


RING = 4
RINGB = 2
RINGI = 4


def _sc_gather(h, dst3, src3):
    mesh = plsc.VectorSubcoreMesh(core_axis_name="c", subcore_axis_name="s")

    @functools.partial(
        pl.kernel,
        out_type=[jax.ShapeDtypeStruct((ECH, H), _bf16)] * 2,
        mesh=mesh,
        compiler_params=pltpu.CompilerParams(needs_layout_passes=False),
        scratch_types=[
            pltpu.VMEM((NCHUNK, CH), jnp.int32),
            pltpu.VMEM((NCHUNK, CH), jnp.int32),
            pltpu.VMEM((RING * CH, H), _f32),
            pltpu.VMEM((RING * CH, H), _f32),
            pltpu.VMEM((RINGB * CH, H), _bf16),
            pltpu.VMEM((RINGB * CH, H), _bf16),
            [pltpu.SemaphoreType.DMA] * RING,
            [pltpu.SemaphoreType.DMA] * RINGB,
        ],
    )
    def k(h_hbm, dst_hbm, src_hbm, hd_hbm, hs_hbm, dsti, srci, bufd, bufs,
          bufd16, bufs16, semg, semw):
        wid = lax.axis_index("s") * NC + lax.axis_index("c")
        pltpu.sync_copy(dst_hbm.at[wid], dsti)
        pltpu.sync_copy(src_hbm.at[wid], srci)

        def start_gather(j, s):
            off = s * CH
            pltpu.async_copy(h_hbm.at[dsti.at[j]], bufd.at[pl.ds(off, CH)],
                             semg[s])
            pltpu.async_copy(h_hbm.at[srci.at[j]], bufs.at[pl.ds(off, CH)],
                             semg[s])

        def drain_gather(s):
            off = s * CH
            pltpu.make_async_copy(h_hbm.at[pl.ds(0, CH)],
                                  bufd.at[pl.ds(off, CH)], semg[s]).wait()
            pltpu.make_async_copy(h_hbm.at[pl.ds(0, CH)],
                                  bufs.at[pl.ds(off, CH)], semg[s]).wait()

        def start_wb16(j, sb):
            off = sb * CH
            base = wid * EPW + j * CH
            pltpu.async_copy(bufd16.at[pl.ds(off, CH)],
                             hd_hbm.at[pl.ds(base, CH)], semw[sb])
            pltpu.async_copy(bufs16.at[pl.ds(off, CH)],
                             hs_hbm.at[pl.ds(base, CH)], semw[sb])

        def drain_wb16(sb):
            off = sb * CH
            pltpu.make_async_copy(hd_hbm.at[pl.ds(0, CH)],
                                  bufd16.at[pl.ds(off, CH)], semw[sb]).wait()
            pltpu.make_async_copy(hd_hbm.at[pl.ds(0, CH)],
                                  bufs16.at[pl.ds(off, CH)], semw[sb]).wait()

        def convert(s, sb):
            offf = s * CH
            offb = sb * CH

            def row(r, carry):
                for g in range(H // 32):
                    a = bufd[offf + r, pl.ds(g * 32, 16)]
                    b = bufd[offf + r, pl.ds(g * 32 + 16, 16)]
                    bufd16[offb + r, pl.ds(g * 32, 32)] = plsc.pack(
                        a, b, format=plsc.PackFormat.INTERLEAVED)
                    a2 = bufs[offf + r, pl.ds(g * 32, 16)]
                    b2 = bufs[offf + r, pl.ds(g * 32 + 16, 16)]
                    bufs16[offb + r, pl.ds(g * 32, 32)] = plsc.pack(
                        a2, b2, format=plsc.PackFormat.INTERLEAVED)
                return carry

            lax.fori_loop(0, CH, row, 0, unroll=False)

        def stage(j, s, sb):
            @pl.when(j <= NCHUNK - 3)
            def _():
                start_gather(j + 2, (s + 2) % RING)

            drain_gather(s)

            @pl.when(j >= 2)
            def _():
                drain_wb16(sb)

            convert(s, sb)
            start_wb16(j, sb)

        start_gather(0, 0)
        start_gather(1, 1)

        def body(jj, carry):
            for u in range(RING):
                stage(jj * RING + u, u, u % RINGB)
            return carry

        lax.fori_loop(0, NCHUNK // RING, body, 0, unroll=False)
        stage(NCHUNK - 1, (NCHUNK - 1) % RING, (NCHUNK - 1) % RINGB)
        for jt in range(NCHUNK - RINGB, NCHUNK):
            drain_wb16(jt % RINGB)

    return k(h, dst3, src3)


def _sc_scatter(msg, dst, init):
    mesh = plsc.VectorSubcoreMesh(core_axis_name="c", subcore_axis_name="s")

    @functools.partial(
        pl.kernel,
        out_type=jax.ShapeDtypeStruct((NC, NPAD, H), _f32),
        mesh=mesh,
        compiler_params=pltpu.CompilerParams(needs_layout_passes=False),
        scratch_types=[
            pltpu.VMEM((RINGI, CH), jnp.int32),
            pltpu.VMEM((RINGB * CH, H), _bf16),
            pltpu.VMEM((RINGB * CH, H), _f32),
            pltpu.VMEM_SHARED((NPAD, H), _f32),
            [pltpu.SemaphoreType.DMA] * RINGB,
            [pltpu.SemaphoreType.DMA] * RINGB,
        ],
    )
    def k(msg_hbm, dst_hbm, init_hbm, out_hbm, idxr, bufb, buff, acc, seml,
          sema):
        cid = lax.axis_index("c")
        sid = lax.axis_index("s")
        wid = sid * NC + cid
        row0 = sid * ROWS_PER_SUB
        pltpu.sync_copy(init_hbm.at[cid, pl.ds(row0, ROWS_PER_SUB)],
                        acc.at[pl.ds(row0, ROWS_PER_SUB)])
        plsc.subcore_barrier()

        def start_load(j, sb):
            base = wid * EPW + j * CH
            pltpu.async_copy(msg_hbm.at[pl.ds(base, CH)],
                             bufb.at[pl.ds(sb * CH, CH)], seml[sb])
            pltpu.async_copy(dst_hbm.at[wid, j], idxr.at[j % RINGI],
                             seml[sb])

        def drain_load(sb):
            pltpu.make_async_copy(msg_hbm.at[pl.ds(0, CH)],
                                  bufb.at[pl.ds(sb * CH, CH)],
                                  seml[sb]).wait()
            pltpu.make_async_copy(dst_hbm.at[0, 0], idxr.at[0],
                                  seml[sb]).wait()

        def drain_add(sb):
            pltpu.make_async_copy(init_hbm.at[0, pl.ds(0, CH)],
                                  buff.at[pl.ds(sb * CH, CH)],
                                  sema[sb]).wait()

        def convert(sb):
            offb = sb * CH

            def row(r, carry):
                for g in range(H // 32):
                    ab = bufb[offb + r, pl.ds(g * 32, 32)]
                    a, b = plsc.unpack(ab, format=plsc.PackFormat.INTERLEAVED)
                    buff[offb + r, pl.ds(g * 32, 16)] = a
                    buff[offb + r, pl.ds(g * 32 + 16, 16)] = b
                return carry

            lax.fori_loop(0, CH, row, 0, unroll=False)

        def stage(j, sb, si):
            drain_load(sb)

            @pl.when(j >= 2)
            def _():
                drain_add(sb)

            convert(sb)
            pltpu.async_copy(buff.at[pl.ds(sb * CH, CH)], acc.at[idxr.at[si]],
                             sema[sb], add=True)

            @pl.when(j <= NCHUNK - 3)
            def _():
                start_load(j + 2, sb)

        start_load(0, 0)
        start_load(1, 1)

        def body(jj, carry):
            for u in range(RINGI):
                j = jj * RINGI + u
                stage(j, u % RINGB, u)
            return carry

        lax.fori_loop(0, NCHUNK // RINGI, body, 0, unroll=False)
        stage(NCHUNK - 1, (NCHUNK - 1) % RINGB, (NCHUNK - 1) % RINGI)
        for jt in range(NCHUNK - RINGB, NCHUNK):
            drain_add(jt % RINGB)
        plsc.subcore_barrier()
        pltpu.sync_copy(acc.at[pl.ds(row0, ROWS_PER_SUB)],
                        out_hbm.at[cid, pl.ds(row0, ROWS_PER_SUB)])

    return k(msg, dst, init)


def kernel(x, edge_attr, edge_index, params):
    src = edge_index[0]
    dst = edge_index[1]
    dst3c = [lax.slice(dst, (c * ECH,), ((c + 1) * ECH,)).reshape(
        NW, NCHUNK, CH) for c in range(K)]
    src3c = [lax.slice(src, (c * ECH,), ((c + 1) * ECH,)).reshape(
        NW, NCHUNK, CH) for c in range(K)]
    zeros_acc = jnp.zeros((NC, NPAD, H), _f32)

    def b2(v):
        return v.reshape(1, -1)

    def wb(w):
        return w.astype(_bf16)

    enc = params["node_enc"]
    eenc = params["edge_enc"]
    gnn = params["gnn"]
    dec = params["decoder"]

    h = _node_enc(x, enc["W0"], b2(enc["b0"]), enc["W1"], b2(enc["b1"]))

    e_chunks = []
    for l in range(2):
        p = gnn[l]
        em, nm = p["edge_mlp"], p["node_mlp"]
        wargs = (wb(em["W0"][0:H])[_PERM], wb(em["W0"][H:2 * H])[_PERM],
                 wb(nm["W0"][0:H])[_PERM], wb(em["W0"][2 * H:3 * H]),
                 b2(em["b0"]), wb(em["W1"]), b2(em["b1"]),
                 wb(nm["W0"][H:2 * H]), b2(nm["b0"]),
                 wb(nm["W1"])[:, _PERM], b2(nm["b1"])[:, _PERM])
        gathered = [_sc_gather(h, dst3c[c], src3c[c]) for c in range(K)]
        s_parts = zeros_acc
        for c in range(K):
            hd, hs = gathered[c]
            if l == 0:
                ec, msg = _edge_l0(
                    c, edge_attr, hd, hs,
                    wb(eenc["W0"]), b2(eenc["b0"]), wb(eenc["W1"]),
                    b2(eenc["b1"]), *wargs, b2(p["en_g"]), b2(p["en_b"]))
                e_chunks.append(ec)
            else:
                msg = _edge_l1(e_chunks[c], hd, hs, *wargs)
            s_parts = _sc_scatter(msg, dst3c[c], s_parts)
        if l == 0:
            h = _node_update(h, s_parts, b2(p["xn_g"]), b2(p["xn_b"]))
        else:
            out = _node_decode(h, s_parts, b2(p["xn_g"]), b2(p["xn_b"]),
                               dec["W0"], b2(dec["b0"]), dec["W1"],
                               b2(dec["b1"]))
    return out


# speedup vs baseline: 1.1681x; 1.1681x over previous
import functools

import jax
import jax.numpy as jnp
from jax import lax
from jax.experimental import pallas as pl
from jax.experimental.pallas import tpu as pltpu
from jax.experimental.pallas import tpu_sc as plsc

N = 10000
E = 320000
H = 128
OUT_DIM = 2

NC = 2
NS = 16
NW = NC * NS
CH = 80
ROWS_PER_SUB = 632
NPAD = NS * ROWS_PER_SUB

K = 5
ECH = E // K
EPW = ECH // NW
NCHUNK = EPW // CH

NB = 1000
EB = 4000

_f32 = jnp.float32
_bf16 = jnp.bfloat16


def _ln(x, g, b):
    m = jnp.mean(x, axis=-1, keepdims=True)
    v = jnp.mean((x - m) ** 2, axis=-1, keepdims=True)
    return (x - m) * lax.rsqrt(v + 1e-5) * g + b


def _dot(a, b):
    return jnp.dot(a, b, preferred_element_type=_f32)


def _bdot(a, b):
    return jnp.dot(a.astype(_bf16), b, preferred_element_type=_f32)


def _node_enc_body(x, w0, b0, w1, b1, h_o):
    h_o[...] = _dot(jax.nn.relu(_dot(x[...], w0[...]) + b0[...]), w1[...]) + b1[...]


def _node_enc(x, w0, b0, w1, b1):
    grid = N // NB
    blk = pl.BlockSpec((NB, H), lambda i: (i, 0))
    wspec = pl.BlockSpec((H, H), lambda i: (0, 0))
    bspec = pl.BlockSpec((1, H), lambda i: (0, 0))
    return pl.pallas_call(
        _node_enc_body,
        grid=(grid,),
        in_specs=[blk, wspec, bspec, wspec, bspec],
        out_specs=blk,
        out_shape=jax.ShapeDtypeStruct((N, H), _f32),
    )(x, w0, b0, w1, b1)


def _edge_core(e0, hdb, hsb, wa, wb, wd, wc, be0, we1, be1, wg, bn0, wn1,
               bn1):
    u = (_dot(hdb, wa[...]) + _dot(hsb, wb[...]) + _bdot(e0, wc[...])
         + be0[...])
    e_new = _bdot(jax.nn.relu(u), we1[...]) + be1[...]
    v = _dot(hdb, wd[...]) + _bdot(e_new, wg[...]) + bn0[...]
    msg = _bdot(jax.nn.relu(v), wn1[...]) + bn1[...]
    return e_new, msg


def _edge_body_l0(ea, hd, hs, ew0, eb0, ew1, eb1, wa, wb, wd, wc, be0, we1,
                  be1, wg, bn0, wn1, bn1, eng, enb, e_o, msg_o):
    e0 = _bdot(jax.nn.relu(_bdot(ea[...], ew0[...]) + eb0[...]), ew1[...]) \
        + eb1[...]
    e_new, msg = _edge_core(e0, hd[...].astype(_bf16), hs[...].astype(_bf16),
                            wa, wb, wd, wc, be0, we1, be1, wg, bn0, wn1, bn1)
    e_o[...] = _ln(e0 + e_new, eng[...], enb[...]).astype(_bf16)
    msg_o[...] = msg


def _edge_body_l1(e, hd, hs, wa, wb, wd, wc, be0, we1, be1, wg, bn0, wn1,
                  bn1, msg_o):
    _, msg = _edge_core(e[...].astype(_f32), hd[...].astype(_bf16),
                        hs[...].astype(_bf16), wa, wb, wd, wc, be0, we1, be1,
                        wg, bn0, wn1, bn1)
    msg_o[...] = msg


def _edge_l0(c, ea_full, hd, hs, ew0, eb0, ew1, eb1, wa, wb, wd, wc, be0,
             we1, be1, wg, bn0, wn1, bn1, eng, enb):
    grid = ECH // EB
    d_edge = ea_full.shape[1]
    eablk = pl.BlockSpec((EB, d_edge), lambda i: (c * (ECH // EB) + i, 0))
    eblk = pl.BlockSpec((EB, H), lambda i: (i, 0))
    wspec = pl.BlockSpec((H, H), lambda i: (0, 0))
    bspec = pl.BlockSpec((1, H), lambda i: (0, 0))
    return pl.pallas_call(
        _edge_body_l0,
        grid=(grid,),
        in_specs=[eablk, eblk, eblk,
                  pl.BlockSpec((d_edge, H), lambda i: (0, 0)), bspec,
                  wspec, bspec,
                  wspec, wspec, wspec, wspec, bspec, wspec, bspec, wspec,
                  bspec, wspec, bspec, bspec, bspec],
        out_specs=[eblk, eblk],
        out_shape=[jax.ShapeDtypeStruct((ECH, H), _bf16),
                   jax.ShapeDtypeStruct((ECH, H), _f32)],
    )(ea_full, hd, hs, ew0, eb0, ew1, eb1, wa, wb, wd, wc, be0, we1, be1,
      wg, bn0, wn1, bn1, eng, enb)


def _edge_l1(e, hd, hs, wa, wb, wd, wc, be0, we1, be1, wg, bn0, wn1, bn1):
    grid = ECH // EB
    eblk = pl.BlockSpec((EB, H), lambda i: (i, 0))
    wspec = pl.BlockSpec((H, H), lambda i: (0, 0))
    bspec = pl.BlockSpec((1, H), lambda i: (0, 0))
    return pl.pallas_call(
        _edge_body_l1,
        grid=(grid,),
        in_specs=[eblk, eblk, eblk, wspec, wspec, wspec, wspec, bspec,
                  wspec, bspec, wspec, bspec, wspec, bspec],
        out_specs=eblk,
        out_shape=jax.ShapeDtypeStruct((ECH, H), _f32),
    )(e, hd, hs, wa, wb, wd, wc, be0, we1, be1, wg, bn0, wn1, bn1)


def _node_upd_body(h, s0, s1, g, b, h_o):
    h_o[...] = _ln(h[...] + s0[0] + s1[0], g[...], b[...])


def _node_update(h, s_parts, g, b):
    grid = N // NB
    blk = pl.BlockSpec((NB, H), lambda i: (i, 0))
    sblk = pl.BlockSpec((1, NB, H), lambda i: (0, i, 0))
    bspec = pl.BlockSpec((1, H), lambda i: (0, 0))
    return pl.pallas_call(
        _node_upd_body,
        grid=(grid,),
        in_specs=[blk, sblk, sblk, bspec, bspec],
        out_specs=blk,
        out_shape=jax.ShapeDtypeStruct((N, H), _f32),
    )(h, s_parts[0:1], s_parts[1:2], g, b)


def _node_dec_body(h, s0, s1, g, b, w0, b0, w1, b1, out_o):
    hn = _ln(h[...] + s0[0] + s1[0], g[...], b[...])
    out_o[...] = _dot(jax.nn.relu(_dot(hn, w0[...]) + b0[...]), w1[...]) + b1[...]


def _node_decode(h, s_parts, g, b, w0, b0, w1, b1):
    grid = N // NB
    blk = pl.BlockSpec((NB, H), lambda i: (i, 0))
    sblk = pl.BlockSpec((1, NB, H), lambda i: (0, i, 0))
    wspec = pl.BlockSpec((H, H), lambda i: (0, 0))
    bspec = pl.BlockSpec((1, H), lambda i: (0, 0))
    return pl.pallas_call(
        _node_dec_body,
        grid=(grid,),
        in_specs=[blk, sblk, sblk, bspec, bspec, wspec, bspec,
                  pl.BlockSpec((H, OUT_DIM), lambda i: (0, 0)),
                  pl.BlockSpec((1, OUT_DIM), lambda i: (0, 0))],
        out_specs=pl.BlockSpec((NB, OUT_DIM), lambda i: (i, 0)),
        out_shape=jax.ShapeDtypeStruct((N, OUT_DIM), _f32),
    )(h, s_parts[0:1], s_parts[1:2], g, b, w0, b0, w1, b1)


RING = 4


def _sc_gather(h, dst3, src3):
    mesh = plsc.VectorSubcoreMesh(core_axis_name="c", subcore_axis_name="s")

    @functools.partial(
        pl.kernel,
        out_type=[jax.ShapeDtypeStruct((ECH, H), _f32)] * 2,
        mesh=mesh,
        scratch_types=[
            pltpu.VMEM((NCHUNK, CH), jnp.int32),
            pltpu.VMEM((NCHUNK, CH), jnp.int32),
            pltpu.VMEM((RING * CH, H), _f32),
            pltpu.VMEM((RING * CH, H), _f32),
            [pltpu.SemaphoreType.DMA] * RING,
            [pltpu.SemaphoreType.DMA] * RING,
        ],
    )
    def k(h_hbm, dst_hbm, src_hbm, hd_hbm, hs_hbm, dsti, srci, bufd, bufs,
          semg, semw):
        wid = lax.axis_index("s") * NC + lax.axis_index("c")
        pltpu.sync_copy(dst_hbm.at[wid], dsti)
        pltpu.sync_copy(src_hbm.at[wid], srci)

        def start_gather(j, s):
            off = s * CH
            pltpu.async_copy(h_hbm.at[dsti.at[j]], bufd.at[pl.ds(off, CH)],
                             semg[s])
            pltpu.async_copy(h_hbm.at[srci.at[j]], bufs.at[pl.ds(off, CH)],
                             semg[s])

        def drain_gather(s):
            off = s * CH
            pltpu.make_async_copy(hd_hbm.at[pl.ds(0, CH)],
                                  bufd.at[pl.ds(off, CH)], semg[s]).wait()
            pltpu.make_async_copy(hd_hbm.at[pl.ds(0, CH)],
                                  bufs.at[pl.ds(off, CH)], semg[s]).wait()

        def start_writeback(j, s):
            off = s * CH
            base = wid * EPW + j * CH
            pltpu.async_copy(bufd.at[pl.ds(off, CH)],
                             hd_hbm.at[pl.ds(base, CH)], semw[s])
            pltpu.async_copy(bufs.at[pl.ds(off, CH)],
                             hs_hbm.at[pl.ds(base, CH)], semw[s])

        def drain_writeback(s):
            off = s * CH
            pltpu.make_async_copy(hd_hbm.at[pl.ds(0, CH)],
                                  bufd.at[pl.ds(off, CH)], semw[s]).wait()
            pltpu.make_async_copy(hd_hbm.at[pl.ds(0, CH)],
                                  bufs.at[pl.ds(off, CH)], semw[s]).wait()

        def stage(j, s):
            s2 = (s + 2) % RING

            @pl.when(j <= NCHUNK - 3)
            def _():
                @pl.when(j >= 2)
                def _():
                    drain_writeback(s2)
                start_gather(j + 2, s2)

            drain_gather(s)
            start_writeback(j, s)

        start_gather(0, 0)
        start_gather(1, 1)

        def body(jj, carry):
            for s in range(RING):
                stage(jj * RING + s, s)
            return carry

        lax.fori_loop(0, NCHUNK // RING, body, 0, unroll=False)
        stage(NCHUNK - 1, (NCHUNK - 1) % RING)
        for jt in range(NCHUNK - RING, NCHUNK):
            drain_writeback(jt % RING)

    return k(h, dst3, src3)


def _sc_scatter(msg, dst, init):
    mesh = plsc.VectorSubcoreMesh(core_axis_name="c", subcore_axis_name="s")

    @functools.partial(
        pl.kernel,
        out_type=jax.ShapeDtypeStruct((NC, NPAD, H), _f32),
        mesh=mesh,
        scratch_types=[
            pltpu.VMEM((RING, CH), jnp.int32),
            pltpu.VMEM((RING * CH, H), _f32),
            pltpu.VMEM_SHARED((NPAD, H), _f32),
            [pltpu.SemaphoreType.DMA] * RING,
            [pltpu.SemaphoreType.DMA] * RING,
        ],
    )
    def k(msg_hbm, dst_hbm, init_hbm, out_hbm, idxr, bufm, acc, seml, sema):
        cid = lax.axis_index("c")
        sid = lax.axis_index("s")
        wid = sid * NC + cid
        row0 = sid * ROWS_PER_SUB
        pltpu.sync_copy(init_hbm.at[cid, pl.ds(row0, ROWS_PER_SUB)],
                        acc.at[pl.ds(row0, ROWS_PER_SUB)])
        plsc.subcore_barrier()

        def start_load(j, s):
            base = wid * EPW + j * CH
            pltpu.async_copy(msg_hbm.at[pl.ds(base, CH)],
                             bufm.at[pl.ds(s * CH, CH)], seml[s])
            pltpu.async_copy(dst_hbm.at[wid, j], idxr.at[s], seml[s])

        def drain_load(s):
            pltpu.make_async_copy(msg_hbm.at[pl.ds(0, CH)],
                                  bufm.at[pl.ds(s * CH, CH)], seml[s]).wait()
            pltpu.make_async_copy(dst_hbm.at[0, 0], idxr.at[s],
                                  seml[s]).wait()

        def drain_add(s):
            pltpu.make_async_copy(msg_hbm.at[pl.ds(0, CH)],
                                  bufm.at[pl.ds(s * CH, CH)], sema[s]).wait()

        def stage(j, s):
            s2 = (s + 2) % RING

            @pl.when(j <= NCHUNK - 3)
            def _():
                @pl.when(j >= 2)
                def _():
                    drain_add(s2)
                start_load(j + 2, s2)

            drain_load(s)
            pltpu.async_copy(bufm.at[pl.ds(s * CH, CH)], acc.at[idxr.at[s]],
                             sema[s], add=True)

        start_load(0, 0)
        start_load(1, 1)

        def body(jj, carry):
            for s in range(RING):
                stage(jj * RING + s, s)
            return carry

        lax.fori_loop(0, NCHUNK // RING, body, 0, unroll=False)
        stage(NCHUNK - 1, (NCHUNK - 1) % RING)
        for jt in range(NCHUNK - RING, NCHUNK):
            drain_add(jt % RING)
        plsc.subcore_barrier()
        pltpu.sync_copy(acc.at[pl.ds(row0, ROWS_PER_SUB)],
                        out_hbm.at[cid, pl.ds(row0, ROWS_PER_SUB)])

    return k(msg, dst, init)


def kernel(x, edge_attr, edge_index, params):
    src = edge_index[0]
    dst = edge_index[1]
    dst3c = [lax.slice(dst, (c * ECH,), ((c + 1) * ECH,)).reshape(
        NW, NCHUNK, CH) for c in range(K)]
    src3c = [lax.slice(src, (c * ECH,), ((c + 1) * ECH,)).reshape(
        NW, NCHUNK, CH) for c in range(K)]
    zeros_acc = jnp.zeros((NC, NPAD, H), _f32)

    def b2(v):
        return v.reshape(1, -1)

    def wb(w):
        return w.astype(_bf16)

    enc = params["node_enc"]
    eenc = params["edge_enc"]
    gnn = params["gnn"]
    dec = params["decoder"]

    h = _node_enc(x, enc["W0"], b2(enc["b0"]), enc["W1"], b2(enc["b1"]))

    e_chunks = []
    for l in range(2):
        p = gnn[l]
        em, nm = p["edge_mlp"], p["node_mlp"]
        wargs = (wb(em["W0"][0:H]), wb(em["W0"][H:2 * H]),
                 wb(nm["W0"][0:H]), wb(em["W0"][2 * H:3 * H]),
                 b2(em["b0"]), wb(em["W1"]), b2(em["b1"]),
                 wb(nm["W0"][H:2 * H]), b2(nm["b0"]),
                 wb(nm["W1"]), b2(nm["b1"]))
        gathered = [_sc_gather(h, dst3c[c], src3c[c]) for c in range(K)]
        s_parts = zeros_acc
        for c in range(K):
            hd, hs = gathered[c]
            if l == 0:
                ec, msg = _edge_l0(
                    c, edge_attr, hd, hs,
                    wb(eenc["W0"]), b2(eenc["b0"]), wb(eenc["W1"]),
                    b2(eenc["b1"]), *wargs, b2(p["en_g"]), b2(p["en_b"]))
                e_chunks.append(ec)
            else:
                msg = _edge_l1(e_chunks[c], hd, hs, *wargs)
            s_parts = _sc_scatter(msg, dst3c[c], s_parts)
        if l == 0:
            h = _node_update(h, s_parts, b2(p["xn_g"]), b2(p["xn_b"]))
        else:
            out = _node_decode(h, s_parts, b2(p["xn_g"]), b2(p["xn_b"]),
                               dec["W0"], b2(dec["b0"]), dec["W1"],
                               b2(dec["b1"]))
    return out
